# baseline (speedup 1.0000x reference)
import functools

import jax
import jax.numpy as jnp
from jax import lax
from jax.experimental import pallas as pl
from jax.experimental.pallas import tpu as pltpu
from jax.experimental.pallas import tpu_sc as plsc

_INFO = plsc.get_sparse_core_info()
_NC = _INFO.num_cores
_NS = _INFO.num_subcores
_NW = _NC * _NS
_L = _INFO.num_lanes


def _sc_gather(table, idx0, idx1, seq_len):
    B = idx0.shape[0]
    D = table.shape[1]
    b_per_w = B // _NW
    mesh = plsc.VectorSubcoreMesh(core_axis_name="c", subcore_axis_name="s")

    @functools.partial(
        pl.kernel,
        mesh=mesh,
        out_type=jax.ShapeDtypeStruct((B, D), jnp.float32),
        scratch_types=[
            pltpu.VMEM((b_per_w,), jnp.int32),
            pltpu.VMEM((b_per_w,), jnp.int32),
            pltpu.VMEM((b_per_w, D), jnp.float32),
            pltpu.SemaphoreType.DMA,
        ],
    )
    def gather_kernel(table_hbm, i0_hbm, i1_hbm, out_hbm, i0_v, i1_v, rows_v, sem):
        wid = lax.axis_index("s") * _NC + lax.axis_index("c")
        base = wid * b_per_w
        pltpu.sync_copy(i0_hbm.at[pl.ds(base, b_per_w)], i0_v)
        pltpu.sync_copy(i1_hbm.at[pl.ds(base, b_per_w)], i1_v)
        i0_v[...] = i0_v[...] * seq_len + i1_v[...]
        pltpu.async_copy(table_hbm.at[i0_v], rows_v, sem).wait()
        pltpu.sync_copy(rows_v, out_hbm.at[pl.ds(base, b_per_w)])

    return gather_kernel(table, idx0, idx1)


def _tc_head(x, W, b2d, n_blk=128):
    B, D = x.shape
    grid = (D // n_blk,)

    def body(x_ref, w_ref, b_ref, o_ref):
        acc = lax.dot_general(
            x_ref[...], w_ref[...],
            (((1,), (1,)), ((), ())),
            preferred_element_type=jnp.float32,
        )
        o_ref[...] = jnp.tanh(acc + b_ref[...])

    return pl.pallas_call(
        body,
        grid=grid,
        in_specs=[
            pl.BlockSpec((B, D), lambda n: (0, 0)),
            pl.BlockSpec((n_blk, D), lambda n: (n, 0)),
            pl.BlockSpec((1, n_blk), lambda n: (0, n)),
        ],
        out_specs=pl.BlockSpec((B, n_blk), lambda n: (0, n)),
        out_shape=jax.ShapeDtypeStruct((B, D), jnp.float32),
    )(x, W, b2d)


def kernel(hidden_states, cls_indexes, W, b):
    n_batch, seq_len, D = hidden_states.shape
    table = hidden_states.reshape(n_batch * seq_len, D)
    idx = cls_indexes.astype(jnp.int32)
    x = _sc_gather(table, idx[:, 0], idx[:, 1], seq_len)
    return _tc_head(x, W, b.reshape(1, D))


# speedup vs baseline: 2.0197x; 2.0197x over previous
import jax
import jax.numpy as jnp
from jax import lax
from jax.experimental import pallas as pl
from jax.experimental.pallas import tpu as pltpu

_CH = 64


def _fused(table, idx0, idx1, W, b2d, seq_len):
    B = idx0.shape[0]
    D = table.shape[1]
    nch = B // _CH

    def body(i0_ref, i1_ref, table_ref, w_hbm, b_ref, o_ref,
             x_v, w_v, wsem, csems):
        pltpu.make_async_copy(w_hbm, w_v, wsem).start()

        def issue_row(r, _):
            flat = i0_ref[r] * seq_len + i1_ref[r]
            pltpu.make_async_copy(
                table_ref.at[flat], x_v.at[r], csems.at[r // _CH]
            ).start()
            return _

        lax.fori_loop(0, B, issue_row, 0, unroll=8)

        pltpu.make_async_copy(w_hbm, w_v, wsem).wait()
        for c in range(nch):
            sl = pl.ds(c * _CH, _CH)
            pltpu.make_async_copy(
                table_ref.at[pl.ds(0, _CH)], x_v.at[sl], csems.at[c]
            ).wait()
            acc = lax.dot_general(
                x_v[sl, :], w_v[...],
                (((1,), (1,)), ((), ())),
                preferred_element_type=jnp.float32,
            )
            o_ref[sl, :] = jnp.tanh(acc + b_ref[...])

    return pl.pallas_call(
        body,
        in_specs=[
            pl.BlockSpec(memory_space=pltpu.SMEM),
            pl.BlockSpec(memory_space=pltpu.SMEM),
            pl.BlockSpec(memory_space=pltpu.HBM),
            pl.BlockSpec(memory_space=pltpu.HBM),
            pl.BlockSpec(memory_space=pltpu.VMEM),
        ],
        out_specs=pl.BlockSpec(memory_space=pltpu.VMEM),
        out_shape=jax.ShapeDtypeStruct((B, D), jnp.float32),
        scratch_shapes=[
            pltpu.VMEM((B, D), jnp.float32),
            pltpu.VMEM((D, D), jnp.float32),
            pltpu.SemaphoreType.DMA,
            pltpu.SemaphoreType.DMA((nch,)),
        ],
    )(idx0, idx1, table, W, b2d)


def kernel(hidden_states, cls_indexes, W, b):
    n_batch, seq_len, D = hidden_states.shape
    table = hidden_states.reshape(n_batch * seq_len, D)
    idx = cls_indexes.astype(jnp.int32)
    return _fused(table, idx[:, 0], idx[:, 1], W, b.reshape(1, D), seq_len)
